# baseline (speedup 1.0000x reference)
import functools

import jax
import jax.numpy as jnp
from jax import lax
from jax.experimental import pallas as pl
from jax.experimental.pallas import tpu as pltpu
from jax.experimental.pallas import tpu_sc as plsc

N_NODES = 10000
N_EDGES = 320000
D_FEAT = 128
LANES = 16

NUM_CORES = 2
NUM_SUBCORES = 16
NUM_WORKERS = NUM_CORES * NUM_SUBCORES
EDGES_PER_WORKER = N_EDGES // NUM_WORKERS
CHUNK = 80
NUM_CHUNKS = EDGES_PER_WORKER // CHUNK
NBUF = 5
NUM_GROUPS = NUM_CHUNKS // NBUF


def _make_score_kernel():
    mesh = plsc.VectorSubcoreMesh(core_axis_name="c", subcore_axis_name="s")

    scratch = (
        [pltpu.VMEM((CHUNK,), jnp.int32) for _ in range(NBUF)]
        + [pltpu.VMEM((CHUNK,), jnp.int32) for _ in range(NBUF)]
        + [pltpu.VMEM((CHUNK, D_FEAT), jnp.float32) for _ in range(NBUF)]
        + [pltpu.VMEM((CHUNK, D_FEAT), jnp.float32) for _ in range(NBUF)]
        + [pltpu.VMEM((CHUNK,), jnp.float32) for _ in range(NBUF)]
        + [pltpu.SemaphoreType.DMA for _ in range(3 * NBUF)]
    )

    @functools.partial(
        pl.kernel,
        out_type=jax.ShapeDtypeStruct((N_EDGES,), jnp.float32),
        mesh=mesh,
        compiler_params=pltpu.CompilerParams(needs_layout_passes=False),
        scratch_types=scratch,
    )
    def score_kernel(x_hbm, src_hbm, dst_hbm, out_hbm, *sc):
        sidx = sc[0:NBUF]
        didx = sc[NBUF:2 * NBUF]
        srows = sc[2 * NBUF:3 * NBUF]
        drows = sc[3 * NBUF:4 * NBUF]
        outv = sc[4 * NBUF:5 * NBUF]
        isem = sc[5 * NBUF:6 * NBUF]
        gsem = sc[6 * NBUF:7 * NBUF]
        osem = sc[7 * NBUF:8 * NBUF]

        wid = lax.axis_index("s") * NUM_CORES + lax.axis_index("c")
        worker_base = wid * EDGES_PER_WORKER
        lane = lax.iota(jnp.int32, LANES)
        last_lane = lane == (LANES - 1)

        def fire_idx(i, b):
            base = worker_base + i * CHUNK
            pltpu.async_copy(src_hbm.at[pl.ds(base, CHUNK)], sidx[b], isem[b])
            pltpu.async_copy(dst_hbm.at[pl.ds(base, CHUNK)], didx[b], isem[b])

        def wait_idx(b):
            pltpu.make_async_copy(src_hbm.at[pl.ds(0, CHUNK)], sidx[b], isem[b]).wait()
            pltpu.make_async_copy(dst_hbm.at[pl.ds(0, CHUNK)], didx[b], isem[b]).wait()

        def fire_gather(b):
            pltpu.async_copy(x_hbm.at[sidx[b]], srows[b], gsem[b])
            pltpu.async_copy(x_hbm.at[didx[b]], drows[b], gsem[b])

        def wait_gather(b):
            pltpu.make_async_copy(x_hbm.at[sidx[b]], srows[b], gsem[b]).wait()
            pltpu.make_async_copy(x_hbm.at[didx[b]], drows[b], gsem[b]).wait()

        def compute(b):
            sr, dr, ov = srows[b], drows[b], outv[b]

            @plsc.parallel_loop(0, CHUNK, unroll=4)
            def _edge_body(e):
                prods = [sr[e, pl.ds(j * LANES, LANES)]
                         * dr[e, pl.ds(j * LANES, LANES)]
                         for j in range(D_FEAT // LANES)]
                while len(prods) > 1:
                    prods = [a + b_ for a, b_ in zip(prods[::2], prods[1::2])]
                cum = plsc.cumsum(prods[0])
                plsc.store_scatter(ov, [jnp.full((LANES,), e, jnp.int32)],
                                   cum, mask=last_lane)

        def fire_out(i, b):
            base = worker_base + i * CHUNK
            pltpu.async_copy(outv[b], out_hbm.at[pl.ds(base, CHUNK)], osem[b])

        def wait_out(b):
            pltpu.make_async_copy(outv[b], out_hbm.at[pl.ds(0, CHUNK)], osem[b]).wait()

        GDEPTH = 3

        def body(i, b, do_fire_idx, do_fire_gather, do_wait_out):
            if do_fire_idx:
                fire_idx(i + (NBUF - 1), (b + NBUF - 1) % NBUF)
            if do_fire_gather:
                wait_idx((b + GDEPTH) % NBUF)
                fire_gather((b + GDEPTH) % NBUF)
            wait_gather(b)
            if do_wait_out:
                wait_out(b)
            compute(b)
            fire_out(i, b)

        for b in range(NBUF - 1):
            fire_idx(b, b)
        for b in range(GDEPTH):
            wait_idx(b)
            fire_gather(b)

        for b in range(NBUF):
            body(b, b, True, True, False)

        def group_body(g, carry):
            i0 = g * NBUF
            for b in range(NBUF):
                body(i0 + b, b, True, True, True)
            return carry

        lax.fori_loop(1, NUM_GROUPS - 1, group_body, 0)

        i0 = (NUM_GROUPS - 1) * NBUF
        for b in range(NBUF):
            i = i0 + b
            body(i, b, i + NBUF - 1 < NUM_CHUNKS, i + GDEPTH < NUM_CHUNKS, True)

        for b in range(NBUF):
            wait_out(b)

    return score_kernel


_score_kernel = _make_score_kernel()


def kernel(x, edge_index):
    scores = _score_kernel(x, edge_index[0], edge_index[1])
    return scores[:, None]
